# baseline (speedup 1.0000x reference)
import functools

import jax
import jax.numpy as jnp
from jax.experimental import pallas as pl
from jax.experimental.pallas import tpu as pltpu

L = 12
B = 1024
D = 768
T = 8
R = 16
C = 100
TR = T * R
NH = 2
HB = B // NH


def _moe_kernel(batch_ref, w0_ref, v_ref, u_ref, hw_ref, hb_ref, out_ref,
                x_ref, counts_ref):
    l = pl.program_id(0)

    @pl.when(l == 0)
    def _init():
        x_ref[...] = batch_ref[...]
        counts_ref[...] = jnp.zeros_like(counts_ref)

    for h in range(NH):
        rows = slice(h * HB, (h + 1) * HB)
        x = x_ref[rows, :]

        base = jax.lax.dot_general(
            x, w0_ref[0],
            (((1,), (1,)), ((), ())),
            preferred_element_type=jnp.float32,
        )

        projT = jax.lax.dot_general(
            v_ref[0], x,
            (((1,), (1,)), ((), ())),
            preferred_element_type=jnp.float32,
        )
        psqT = projT * projT
        ssqT = jnp.concatenate(
            [jnp.sum(psqT[t * R:(t + 1) * R, :], axis=0, keepdims=True)
             for t in range(T)], axis=0)
        logitsT = jnp.sqrt(ssqT + 1e-12)

        mx = jnp.max(logitsT, axis=0, keepdims=True)
        ridx = jax.lax.broadcasted_iota(jnp.int32, (T, HB), 0)
        selT = jnp.min(jnp.where(logitsT >= mx, ridx, T), axis=0,
                       keepdims=True)

        counts_ref[:, rows] += (ridx == selT).astype(jnp.float32)

        rgrp = jax.lax.broadcasted_iota(jnp.int32, (TR, HB), 0) // R
        maskedT = jnp.where(rgrp == selT, projT, 0.0)
        delta = jax.lax.dot_general(
            maskedT, u_ref[0],
            (((0,), (0,)), ((), ())),
            preferred_element_type=jnp.float32,
        )

        y = base + delta

        @pl.when(l < L - 1)
        def _mid():
            x_ref[rows, :] = jax.nn.gelu(y)

        @pl.when(l == L - 1)
        def _final():
            counts = counts_ref[:, rows].T
            cmx = jnp.max(counts, axis=1, keepdims=True)
            cidx = jax.lax.broadcasted_iota(jnp.int32, (HB, T), 1)
            maj = jnp.min(jnp.where(counts >= cmx, cidx, T), axis=1,
                          keepdims=True)
            head_all = jax.lax.dot_general(
                y, hw_ref[...],
                (((1,), (0,)), ((), ())),
                preferred_element_type=jnp.float32,
            )
            acc = jnp.zeros((HB, C), dtype=jnp.float32)
            for t in range(T):
                hd = head_all[:, t * C:(t + 1) * C] + hb_ref[t:t + 1, :]
                acc = jnp.where(maj == t, hd, acc)
            out_ref[rows, :] = acc


@functools.partial(jax.jit, static_argnames=("interpret",))
def kernel(batch, W0, V, U, heads_W, heads_b, interpret=False):
    V_flat = V.reshape(L, TR, D)
    U_perm = U.transpose(0, 1, 3, 2).reshape(L, TR, D)
    heads_flat = heads_W.transpose(2, 0, 1).reshape(D, T * C)

    out = pl.pallas_call(
        _moe_kernel,
        grid=(L,),
        in_specs=[
            pl.BlockSpec((B, D), lambda l: (0, 0)),
            pl.BlockSpec((1, D, D), lambda l: (l, 0, 0)),
            pl.BlockSpec((1, TR, D), lambda l: (l, 0, 0)),
            pl.BlockSpec((1, TR, D), lambda l: (l, 0, 0)),
            pl.BlockSpec((D, T * C), lambda l: (0, 0)),
            pl.BlockSpec((T, C), lambda l: (0, 0)),
        ],
        out_specs=pl.BlockSpec((B, C), lambda l: (0, 0)),
        out_shape=jax.ShapeDtypeStruct((B, C), jnp.float32),
        scratch_shapes=[
            pltpu.VMEM((B, D), jnp.float32),
            pltpu.VMEM((T, B), jnp.float32),
        ],
        interpret=interpret,
    )(batch, W0, V_flat, U_perm, heads_flat, heads_b)
    return out


# speedup vs baseline: 1.1359x; 1.1359x over previous
import functools

import jax
import jax.numpy as jnp
from jax.experimental import pallas as pl
from jax.experimental.pallas import tpu as pltpu

L = 12
B = 1024
D = 768
T = 8
R = 16
C = 100
TR = T * R


def _moe_kernel(batch_ref, w0_ref, v_ref, u_ref, hw_ref, hb_ref, out_ref,
                x_ref, counts_ref):
    l = pl.program_id(0)

    @pl.when(l == 0)
    def _init():
        x_ref[...] = batch_ref[...]
        counts_ref[...] = jnp.zeros_like(counts_ref)

    x = x_ref[...]

    base = jax.lax.dot_general(
        x, w0_ref[0],
        (((1,), (1,)), ((), ())),
        preferred_element_type=jnp.float32,
    )

    projT = jax.lax.dot_general(
        v_ref[0], x,
        (((1,), (1,)), ((), ())),
        preferred_element_type=jnp.float32,
    )
    psqT = projT * projT
    ssqT = jnp.concatenate(
        [jnp.sum(psqT[t * R:(t + 1) * R, :], axis=0, keepdims=True)
         for t in range(T)], axis=0)
    logitsT = jnp.sqrt(ssqT + 1e-12)

    mx = jnp.max(logitsT, axis=0, keepdims=True)
    ridx = jax.lax.broadcasted_iota(jnp.int32, (T, B), 0)
    selT = jnp.min(jnp.where(logitsT >= mx, ridx, T), axis=0, keepdims=True)

    counts_ref[...] += (ridx == selT).astype(jnp.float32)

    rgrp = jax.lax.broadcasted_iota(jnp.int32, (TR, B), 0) // R
    maskedT = jnp.where(rgrp == selT, projT, 0.0)
    delta = jax.lax.dot_general(
        maskedT, u_ref[0],
        (((0,), (0,)), ((), ())),
        preferred_element_type=jnp.float32,
    )

    y = base + delta

    @pl.when(l < L - 1)
    def _mid():
        x_ref[...] = jax.nn.gelu(y)

    @pl.when(l == L - 1)
    def _final():
        counts = counts_ref[...].T
        cmx = jnp.max(counts, axis=1, keepdims=True)
        cidx = jax.lax.broadcasted_iota(jnp.int32, (B, T), 1)
        maj = jnp.min(jnp.where(counts >= cmx, cidx, T), axis=1, keepdims=True)
        head_all = jax.lax.dot_general(
            y, hw_ref[...],
            (((1,), (0,)), ((), ())),
            preferred_element_type=jnp.float32,
        )
        acc = jnp.zeros((B, C), dtype=jnp.float32)
        for t in range(T):
            hd = head_all[:, t * C:(t + 1) * C] + hb_ref[t:t + 1, :]
            acc = jnp.where(maj == t, hd, acc)
        out_ref[...] = acc


@functools.partial(jax.jit, static_argnames=("interpret",))
def kernel(batch, W0, V, U, heads_W, heads_b, interpret=False):
    V_flat = V.reshape(L, TR, D)
    U_perm = U.transpose(0, 1, 3, 2).reshape(L, TR, D)
    heads_flat = heads_W.transpose(2, 0, 1).reshape(D, T * C)

    out = pl.pallas_call(
        _moe_kernel,
        grid=(L,),
        in_specs=[
            pl.BlockSpec((B, D), lambda l: (0, 0)),
            pl.BlockSpec((1, D, D), lambda l: (l, 0, 0)),
            pl.BlockSpec((1, TR, D), lambda l: (l, 0, 0)),
            pl.BlockSpec((1, TR, D), lambda l: (l, 0, 0)),
            pl.BlockSpec((D, T * C), lambda l: (0, 0)),
            pl.BlockSpec((T, C), lambda l: (0, 0)),
        ],
        out_specs=pl.BlockSpec((B, C), lambda l: (0, 0)),
        out_shape=jax.ShapeDtypeStruct((B, C), jnp.float32),
        scratch_shapes=[
            pltpu.VMEM((B, D), jnp.float32),
            pltpu.VMEM((T, B), jnp.float32),
        ],
        interpret=interpret,
    )(batch, W0, V_flat, U_perm, heads_flat, heads_b)
    return out
